# baseline (speedup 1.0000x reference)
import functools

import jax
import jax.numpy as jnp
from jax import lax
from jax.experimental import pallas as pl
from jax.experimental.pallas import tpu as pltpu
from jax.experimental.pallas import tpu_sc as plsc

N_NODES = 10000
N_EDGES = 320000
D_FEAT = 128
DW = D_FEAT // 2

NW = 32
P = 313
NSEG_PAD = NW * P
T = 256
IOFF = 8
POISON = NSEG_PAD
TRASH = P * DW

_mesh = plsc.VectorSubcoreMesh(core_axis_name="c", subcore_axis_name="s")


@functools.partial(
    pl.kernel,
    mesh=_mesh,
    out_type=jax.ShapeDtypeStruct((NSEG_PAD * DW,), jnp.int32),
    scratch_types=[
        pltpu.VMEM((16,), jnp.int32),
        pltpu.VMEM((T + IOFF,), jnp.int32),
        pltpu.VMEM((T + IOFF,), jnp.int32),
        pltpu.VMEM((T * DW,), jnp.int32),
        pltpu.VMEM((T * DW,), jnp.int32),
        pltpu.VMEM(((P + 1) * DW,), jnp.int32),
        pltpu.SemaphoreType.DMA,
        pltpu.SemaphoreType.DMA,
        pltpu.SemaphoreType.DMA,
        pltpu.SemaphoreType.DMA,
    ],
    compiler_params=pltpu.CompilerParams(needs_layout_passes=False),
)
def _seg_max_sc(feat_hbm, ids_hbm, starts_hbm, out_hbm,
                sv, idbuf0, idbuf1, rowbuf0, rowbuf1, outbuf,
                si0, sr0, si1, sr1):
    wid = lax.axis_index("s") * 2 + lax.axis_index("c")
    s0 = wid * P

    pltpu.sync_copy(starts_hbm.at[wid], sv)
    svv = sv[pl.ds(0, 16)]
    e0 = svv[0]
    e1 = svv[1]
    a0 = e0 & -16

    idbufs = (idbuf0, idbuf1)
    rowbufs = (rowbuf0, rowbuf1)
    sems = ((si0, sr0), (si1, sr1))

    zero = jnp.zeros((16,), jnp.int32)

    def zbody(i, _):
        b = i * DW
        for k in range(4):
            outbuf[pl.ds(b + 16 * k, 16)] = zero
        return 0

    lax.fori_loop(0, P + 1, zbody, 0)

    n_tiles = lax.div(e1 - a0 + (T - 1), T)

    def copies(t_eff, b):
        astart = pl.multiple_of(a0 + t_eff * T, 8)
        astart_f = pl.multiple_of(jnp.minimum(astart, N_EDGES - T), 8)
        ci = pltpu.make_async_copy(
            ids_hbm.at[pl.ds(astart, T + IOFF)], idbufs[b], sems[b][0]
        )
        cr = pltpu.make_async_copy(
            feat_hbm.at[pl.ds(astart_f * DW, T * DW)],
            rowbufs[b],
            sems[b][1],
        )
        return ci, cr

    def issue(t_eff, b):
        ci, cr = copies(t_eff, b)
        ci.start()
        cr.start()

    def wait(t_eff, b):
        ci, cr = copies(t_eff, b)
        ci.wait()
        cr.wait()

    consts = [jnp.arange(16, dtype=jnp.int32) + 16 * k for k in range(4)]

    def process(b, t_eff, carry):
        idb = idbufs[b]
        rwb = rowbufs[b]
        astart = a0 + t_eff * T
        dshift = astart - jnp.minimum(astart, N_EDGES - T)
        dsh64 = dshift << 6

        def gbody(g, carry):
            gb = g * 16
            idvec = idb[pl.ds(IOFF + gb, 16)]
            idprev = idb[pl.ds(IOFF + gb - 1, 16)]
            startm = idvec != idprev
            nb = plsc.all_reduce_population_count(startm)[0]

            def fast(c):
                accs, obcur = c
                base = (g << 10) + dsh64
                for j in range(16):
                    bj = jnp.minimum(base + j * 64, (T - 1) * 64)
                    rows = tuple(
                        plsc.bitcast(rwb[pl.ds(bj + 16 * k, 16)],
                                     jnp.bfloat16)
                        for k in range(4)
                    )
                    accs = tuple(
                        jnp.maximum(accs[k], rows[k]) for k in range(4)
                    )
                return accs, obcur

            def single(c):
                accs, obcur = c
                bpos = plsc.all_reduce_ffs(startm)
                bps = bpos[0]
                d = idvec - s0
                loc = jnp.where((d < 0) | (d >= P), P, d)
                obv = loc << 6
                base = (g << 10) + dsh64
                flush = accs
                for j in range(16):
                    mj = j == bps
                    bj = jnp.minimum(base + j * 64, (T - 1) * 64)
                    rows = tuple(
                        plsc.bitcast(rwb[pl.ds(bj + 16 * k, 16)],
                                     jnp.bfloat16)
                        for k in range(4)
                    )
                    new_accs = []
                    new_flush = []
                    for k in range(4):
                        new_flush.append(jnp.where(mj, accs[k], flush[k]))
                        m = jnp.maximum(accs[k], rows[k])
                        new_accs.append(jnp.where(mj, rows[k], m))
                    accs = tuple(new_accs)
                    flush = tuple(new_flush)
                for k in range(4):
                    plsc.store_scatter(
                        outbuf,
                        [obcur + consts[k]],
                        plsc.bitcast(flush[k], jnp.int32),
                    )
                obcur = jnp.take_along_axis(obv, bpos, axis=0,
                                            mode="promise_in_bounds")
                return accs, obcur

            def slow(c):
                accs, obcur = c
                for k in range(4):
                    plsc.store_scatter(
                        outbuf,
                        [obcur + consts[k]],
                        plsc.bitcast(accs[k], jnp.int32),
                    )
                same_i = jnp.where(idvec == idprev, 1, 0).astype(jnp.int32)
                d = idvec - s0
                loc = jnp.where((d < 0) | (d >= P), P, d)
                obv = loc << 6
                for j in range(16):
                    jv = jnp.full((16,), j, dtype=jnp.int32)
                    obj = jnp.take_along_axis(obv, jv, axis=0,
                                              mode="promise_in_bounds")
                    smj = jnp.take_along_axis(same_i, jv, axis=0,
                                              mode="promise_in_bounds")
                    sm = smj[0] != 0
                    rb = jnp.minimum(gb + j + dshift, T - 1) << 6
                    new_accs = []
                    for k in range(4):
                        row = plsc.bitcast(rwb[pl.ds(rb + 16 * k, 16)],
                                           jnp.bfloat16)
                        a = jnp.where(sm, jnp.maximum(accs[k], row), row)
                        plsc.store_scatter(
                            outbuf,
                            [obj + consts[k]],
                            plsc.bitcast(a, jnp.int32),
                        )
                        new_accs.append(a)
                    accs = tuple(new_accs)
                    obcur = obj
                return accs, obcur

            return lax.cond(
                nb == 0,
                fast,
                lambda c: lax.cond(nb == 1, single, slow, c),
                carry,
            )

        return lax.fori_loop(0, T // 16, gbody, carry)

    @pl.when(n_tiles > 0)
    def _():
        issue(0, 0)

    @pl.when(n_tiles > 1)
    def _():
        issue(1, 1)

    def pair_body(pp, carry):
        t0 = pp * 2
        wait(t0, 0)
        carry = process(0, t0, carry)

        @pl.when(t0 + 2 < n_tiles)
        def _():
            issue(t0 + 2, 0)

        t1 = t0 + 1
        wait(t1, 1)
        carry = process(1, t1, carry)

        @pl.when(t1 + 2 < n_tiles)
        def _():
            issue(t1 + 2, 1)

        return carry

    init = (
        tuple(jnp.zeros((32,), jnp.bfloat16) for _ in range(4)),
        jnp.full((16,), TRASH, jnp.int32),
    )
    carry = lax.fori_loop(0, lax.div(n_tiles, 2), pair_body, init)

    def tail(c):
        wait(n_tiles - 1, 0)
        return process(0, n_tiles - 1, c)

    accs, obcur = lax.cond(lax.rem(n_tiles, 2) == 1, tail, lambda c: c, carry)

    for k in range(4):
        plsc.store_scatter(
            outbuf, [obcur + consts[k]], plsc.bitcast(accs[k], jnp.int32)
        )

    pltpu.sync_copy(
        outbuf.at[pl.ds(0, P * DW)],
        out_hbm.at[pl.ds(s0 * DW, P * DW)],
    )


def kernel(feat, unq_inv, coor):
    del coor
    bounds = (jnp.arange(NW + 1) * P).astype(jnp.int32)
    seg = jnp.searchsorted(unq_inv, bounds).astype(jnp.int32)
    starts = (
        jnp.zeros((NW, 16), jnp.int32)
        .at[:, 0].set(seg[:-1])
        .at[:, 1].set(seg[1:])
    )
    ids_ext = jnp.concatenate([
        jnp.full((IOFF,), -1, jnp.int32),
        unq_inv,
        jnp.full((T + IOFF,), POISON, jnp.int32),
    ])
    ua = lax.bitcast_convert_type(feat[:, :DW], jnp.uint32)
    ub_ = lax.bitcast_convert_type(feat[:, DW:], jnp.uint32)
    ra = (ua + 0x7FFF + ((ua >> 16) & 1)) >> 16
    rb = (ub_ + 0x7FFF + ((ub_ >> 16) & 1)) >> 16
    packed = lax.bitcast_convert_type((rb << 16) | ra, jnp.int32)
    out_i32 = _seg_max_sc(packed.reshape(-1), ids_ext, starts)
    ub = lax.bitcast_convert_type(
        out_i32.reshape(NSEG_PAD, DW), jnp.bfloat16
    )
    res = jnp.concatenate([ub[:, :, 0], ub[:, :, 1]], axis=1)
    return res[:N_NODES].astype(jnp.float32)


# speedup vs baseline: 4.1632x; 4.1632x over previous
import functools

import jax
import jax.numpy as jnp
from jax import lax
from jax.experimental import pallas as pl
from jax.experimental.pallas import tpu as pltpu
from jax.experimental.pallas import tpu_sc as plsc

N_NODES = 10000
N_EDGES = 320000
D_FEAT = 128

NW = 32
P = 313
NSEG_PAD = NW * P
T = 256
IOFF = 8
POISON = NSEG_PAD
TRASH = P * D_FEAT

_mesh = plsc.VectorSubcoreMesh(core_axis_name="c", subcore_axis_name="s")


@functools.partial(
    pl.kernel,
    mesh=_mesh,
    out_type=jax.ShapeDtypeStruct((NSEG_PAD * D_FEAT,), jnp.float32),
    scratch_types=[
        pltpu.VMEM((16,), jnp.int32),
        pltpu.VMEM((T + IOFF,), jnp.int32),
        pltpu.VMEM((T + IOFF,), jnp.int32),
        pltpu.VMEM((T * D_FEAT,), jnp.float32),
        pltpu.VMEM((T * D_FEAT,), jnp.float32),
        pltpu.VMEM(((P + 1) * D_FEAT,), jnp.float32),
        pltpu.SemaphoreType.DMA,
        pltpu.SemaphoreType.DMA,
        pltpu.SemaphoreType.DMA,
        pltpu.SemaphoreType.DMA,
    ],
    compiler_params=pltpu.CompilerParams(needs_layout_passes=False),
)
def _seg_max_sc(feat_hbm, ids_hbm, starts_hbm, out_hbm,
                sv, idbuf0, idbuf1, rowbuf0, rowbuf1, outbuf,
                si0, sr0, si1, sr1):
    wid = lax.axis_index("s") * 2 + lax.axis_index("c")
    s0 = wid * P

    pltpu.sync_copy(starts_hbm.at[wid], sv)
    svv = sv[pl.ds(0, 16)]
    e0 = svv[0]
    e1 = svv[1]
    a0 = e0 & -16

    idbufs = (idbuf0, idbuf1)
    rowbufs = (rowbuf0, rowbuf1)
    sems = ((si0, sr0), (si1, sr1))

    zero = jnp.zeros((16,), jnp.float32)

    def zbody(i, _):
        b = i * D_FEAT
        for k in range(8):
            outbuf[pl.ds(b + 16 * k, 16)] = zero
        return 0

    lax.fori_loop(0, P + 1, zbody, 0)

    n_tiles = lax.div(e1 - a0 + (T - 1), T)

    def copies(t_eff, b):
        astart = pl.multiple_of(a0 + t_eff * T, 8)
        astart_f = pl.multiple_of(jnp.minimum(astart, N_EDGES - T), 8)
        ci = pltpu.make_async_copy(
            ids_hbm.at[pl.ds(astart, T + IOFF)], idbufs[b], sems[b][0]
        )
        cr = pltpu.make_async_copy(
            feat_hbm.at[pl.ds(astart_f * D_FEAT, T * D_FEAT)],
            rowbufs[b].at[pl.ds(0, T * D_FEAT)],
            sems[b][1],
        )
        return ci, cr

    def issue(t_eff, b):
        ci, cr = copies(t_eff, b)
        ci.start()
        cr.start()

    def wait(t_eff, b):
        ci, cr = copies(t_eff, b)
        ci.wait()
        cr.wait()

    consts = [jnp.arange(16, dtype=jnp.int32) + 16 * k for k in range(8)]

    def process(b, t_eff, carry):
        idb = idbufs[b]
        rwb = rowbufs[b]
        astart = a0 + t_eff * T
        dshift = astart - jnp.minimum(astart, N_EDGES - T)
        dsh128 = dshift << 7

        def gbody(g, carry):
            gb = g * 16
            idvec = idb[pl.ds(IOFF + gb, 16)]
            idprev = idb[pl.ds(IOFF + gb - 1, 16)]
            smvec = idvec == idprev
            startm = idvec != idprev
            nb = plsc.all_reduce_population_count(startm)[0]

            def fast(c):
                accs, obcur = c
                base = (g << 11) + dsh128
                for j in range(16):
                    bj = jnp.minimum(base + j * 128, (T - 1) * 128)
                    rows = tuple(
                        rwb[pl.ds(bj + 16 * k, 16)]
                        for k in range(8)
                    )
                    accs = tuple(
                        jnp.maximum(accs[k], rows[k]) for k in range(8)
                    )
                return accs, obcur

            def single(c):
                accs, obcur = c
                bpos = plsc.all_reduce_ffs(startm)
                d = idvec - s0
                loc = jnp.where((d < 0) | (d >= P), P, d)
                obv = loc << 7
                base = (g << 11) + dsh128
                flush = accs
                for j in range(16):
                    mj = jnp.full((16,), j, dtype=jnp.int32) == bpos
                    bj = jnp.minimum(base + j * 128, (T - 1) * 128)
                    rows = tuple(
                        rwb[pl.ds(bj + 16 * k, 16)]
                        for k in range(8)
                    )
                    new_accs = []
                    new_flush = []
                    for k in range(8):
                        new_flush.append(jnp.where(mj, accs[k], flush[k]))
                        m = jnp.maximum(accs[k], rows[k])
                        new_accs.append(jnp.where(mj, rows[k], m))
                    accs = tuple(new_accs)
                    flush = tuple(new_flush)
                for k in range(8):
                    plsc.store_scatter(outbuf, [obcur + consts[k]], flush[k])
                obcur = jnp.take_along_axis(obv, bpos, axis=0,
                                            mode="promise_in_bounds")
                return accs, obcur

            def slow(c):
                accs, obcur = c
                for k in range(8):
                    plsc.store_scatter(outbuf, [obcur + consts[k]], accs[k])
                same_i = jnp.where(smvec, 1, 0).astype(jnp.int32)
                d = idvec - s0
                loc = jnp.where((d < 0) | (d >= P), P, d)
                obv = loc << 7
                for j in range(16):
                    jv = jnp.full((16,), j, dtype=jnp.int32)
                    obj = jnp.take_along_axis(obv, jv, axis=0,
                                              mode="promise_in_bounds")
                    smj = jnp.take_along_axis(same_i, jv, axis=0,
                                              mode="promise_in_bounds")
                    smask = smj != 0
                    rb = jnp.minimum(gb + j + dshift, T - 1) << 7
                    new_accs = []
                    for k in range(8):
                        row = rwb[pl.ds(rb + 16 * k, 16)]
                        a = jnp.where(smask, jnp.maximum(accs[k], row), row)
                        plsc.store_scatter(outbuf, [obj + consts[k]], a)
                        new_accs.append(a)
                    accs = tuple(new_accs)
                    obcur = obj
                return accs, obcur

            return lax.cond(
                nb == 0,
                fast,
                lambda c: lax.cond(nb == 1, single, slow, c),
                carry,
            )

        return lax.fori_loop(0, T // 16, gbody, carry)

    @pl.when(n_tiles > 0)
    def _():
        issue(0, 0)

    @pl.when(n_tiles > 1)
    def _():
        issue(1, 1)

    def pair_body(pp, carry):
        t0 = pp * 2
        wait(t0, 0)
        carry = process(0, t0, carry)

        @pl.when(t0 + 2 < n_tiles)
        def _():
            issue(t0 + 2, 0)

        t1 = t0 + 1
        wait(t1, 1)
        carry = process(1, t1, carry)

        @pl.when(t1 + 2 < n_tiles)
        def _():
            issue(t1 + 2, 1)

        return carry

    init = (
        tuple(jnp.zeros((16,), jnp.float32) for _ in range(8)),
        jnp.full((16,), TRASH, jnp.int32),
    )
    carry = lax.fori_loop(0, lax.div(n_tiles, 2), pair_body, init)

    def tail(c):
        wait(n_tiles - 1, 0)
        return process(0, n_tiles - 1, c)

    accs, obcur = lax.cond(lax.rem(n_tiles, 2) == 1, tail, lambda c: c, carry)

    for k in range(8):
        plsc.store_scatter(outbuf, [obcur + consts[k]], accs[k])

    pltpu.sync_copy(
        outbuf.at[pl.ds(0, P * D_FEAT)],
        out_hbm.at[pl.ds(s0 * D_FEAT, P * D_FEAT)],
    )


def kernel(feat, unq_inv, coor):
    del coor
    bounds = (jnp.arange(NW + 1) * P).astype(jnp.int32)
    seg = jnp.searchsorted(unq_inv, bounds).astype(jnp.int32)
    starts = (
        jnp.zeros((NW, 16), jnp.int32)
        .at[:, 0].set(seg[:-1])
        .at[:, 1].set(seg[1:])
    )
    ids_ext = jnp.concatenate([
        jnp.full((IOFF,), -1, jnp.int32),
        unq_inv,
        jnp.full((T + IOFF,), POISON, jnp.int32),
    ])
    out_flat = _seg_max_sc(feat.reshape(-1), ids_ext, starts)
    return out_flat.reshape(NSEG_PAD, D_FEAT)[:N_NODES]
